# Initial kernel scaffold; baseline (speedup 1.0000x reference)
import functools

import jax
import jax.numpy as jnp
from jax import lax
from jax.experimental import pallas as pl
from jax.experimental.pallas import tpu as pltpu
from jax.experimental.pallas import tpu_sc as plsc

NCORES = 2
NSUB = 16
LANES = 16
W = 128
ZR = 1250
BN = 2000


def _mesh():
    return plsc.VectorSubcoreMesh(core_axis_name="c", subcore_axis_name="s")


def _make_deg_kernel(N, E):
    EPC = E // (NCORES * NSUB)
    NW = EPC // W
    TAIL = EPC - NW * W
    SR = N // NSUB

    @functools.partial(
        pl.kernel,
        mesh=_mesh(),
        out_type=jax.ShapeDtypeStruct((NCORES, N, LANES), jnp.float32),
        scratch_types=[
            pltpu.VMEM_SHARED((N, LANES), jnp.float32),
            pltpu.VMEM((ZR, LANES), jnp.float32),
            pltpu.VMEM((W, LANES), jnp.float32),
            pltpu.VMEM((W,), jnp.int32),
        ]
        + ([pltpu.VMEM((TAIL, LANES), jnp.float32),
            pltpu.VMEM((TAIL,), jnp.int32)] if TAIL else []),
    )
    def deg_kernel(dst_hbm, out_hbm, acc, zb, ones, dst_v, *tail_bufs):
        c = lax.axis_index("c")
        s = lax.axis_index("s")

        @pl.loop(0, ZR)
        def _(i):
            zb[i, :] = jnp.zeros((LANES,), jnp.float32)

        @pl.loop(0, W)
        def _(i):
            ones[i, :] = jnp.ones((LANES,), jnp.float32)

        @pl.loop(0, SR // ZR)
        def _(j):
            pltpu.sync_copy(zb, acc.at[pl.ds(s * SR + j * ZR, ZR)])

        plsc.subcore_barrier()

        base = c * (E // NCORES) + s * EPC

        @pl.loop(0, NW)
        def _(w):
            pltpu.sync_copy(dst_hbm.at[pl.ds(base + w * W, W)], dst_v)
            pltpu.sync_copy(ones, acc.at[dst_v], add=True)

        if TAIL:
            ones_t, dst_t = tail_bufs

            @pl.loop(0, TAIL)
            def _(i):
                ones_t[i, :] = jnp.ones((LANES,), jnp.float32)

            pltpu.sync_copy(dst_hbm.at[pl.ds(base + NW * W, TAIL)], dst_t)
            pltpu.sync_copy(ones_t, acc.at[dst_t], add=True)

        plsc.subcore_barrier()
        pltpu.sync_copy(acc.at[pl.ds(s * SR, SR)],
                        out_hbm.at[c].at[pl.ds(s * SR, SR)])

    return deg_kernel


def _make_agg_kernel(N, E):
    EPC = E // NSUB
    NW = EPC // W
    TAIL = EPC - NW * W
    SR = N // NSUB

    @functools.partial(
        pl.kernel,
        mesh=_mesh(),
        out_type=jax.ShapeDtypeStruct((NCORES, N, LANES), jnp.float32),
        scratch_types=[
            pltpu.VMEM_SHARED((N, LANES), jnp.float32),
            pltpu.VMEM((ZR, LANES), jnp.float32),
            pltpu.VMEM((W, LANES), jnp.float32),
            pltpu.VMEM((W,), jnp.int32),
            pltpu.VMEM((W,), jnp.int32),
        ]
        + ([pltpu.VMEM((TAIL, LANES), jnp.float32),
            pltpu.VMEM((TAIL,), jnp.int32),
            pltpu.VMEM((TAIL,), jnp.int32)] if TAIL else []),
    )
    def agg_kernel(x_hbm, src_hbm, dst_hbm, out_hbm,
                   acc, zb, msg, src_v, dst_v, *tail_bufs):
        c = lax.axis_index("c")
        s = lax.axis_index("s")

        @pl.loop(0, ZR)
        def _(i):
            zb[i, :] = jnp.zeros((LANES,), jnp.float32)

        @pl.loop(0, SR // ZR)
        def _(j):
            pltpu.sync_copy(zb, acc.at[pl.ds(s * SR + j * ZR, ZR)])

        plsc.subcore_barrier()

        base = s * EPC

        @pl.loop(0, NW)
        def _(w):
            off = base + w * W
            pltpu.sync_copy(src_hbm.at[pl.ds(off, W)], src_v)
            pltpu.sync_copy(dst_hbm.at[pl.ds(off, W)], dst_v)
            pltpu.sync_copy(x_hbm.at[c].at[src_v], msg)
            pltpu.sync_copy(msg, acc.at[dst_v], add=True)

        if TAIL:
            msg_t, src_t, dst_t = tail_bufs
            off = base + NW * W
            pltpu.sync_copy(src_hbm.at[pl.ds(off, TAIL)], src_t)
            pltpu.sync_copy(dst_hbm.at[pl.ds(off, TAIL)], dst_t)
            pltpu.sync_copy(x_hbm.at[c].at[src_t], msg_t)
            pltpu.sync_copy(msg_t, acc.at[dst_t], add=True)

        plsc.subcore_barrier()
        pltpu.sync_copy(acc.at[pl.ds(s * SR, SR)],
                        out_hbm.at[c].at[pl.ds(s * SR, SR)])

    return agg_kernel


def _dot(a, b):
    return jnp.dot(a, b, preferred_element_type=jnp.float32)


def _encode_body(xc_ref, xs_ref, pw_ref, pb_ref, sw_ref, sb_ref, o_ref):
    o_ref[0] = _dot(xc_ref[...], pw_ref[...]) + pb_ref[...]
    o_ref[1] = _dot(xs_ref[...], sw_ref[...]) + sb_ref[...]


def _combine_body(x_ref, a_ref, d_ref, wl_ref, bl_ref, wr_ref, o_ref):
    wl = wl_ref[...]
    wr = wr_ref[...]
    d = d_ref[0, :, 0:1] + d_ref[1, :, 0:1]
    invd = 1.0 / jnp.maximum(d, 1.0)
    h = (_dot(a_ref[0] * invd, wl[:LANES])
         + _dot(a_ref[1] * invd, wl[LANES:])
         + _dot(x_ref[0], wr[:LANES])
         + _dot(x_ref[1], wr[LANES:])
         + bl_ref[...])
    h = jnp.maximum(h, 0.0)
    o_ref[0] = h[:, :LANES]
    o_ref[1] = h[:, LANES:]


def _final_body(x_ref, a_ref, d_ref, wl_ref, bl_ref, wr_ref, ow_ref, ob_ref,
                o_ref):
    wl = wl_ref[...]
    wr = wr_ref[...]
    d = d_ref[0, :, 0:1] + d_ref[1, :, 0:1]
    invd = 1.0 / jnp.maximum(d, 1.0)
    h = (_dot(a_ref[0] * invd, wl[:LANES])
         + _dot(a_ref[1] * invd, wl[LANES:])
         + _dot(x_ref[0], wr[:LANES])
         + _dot(x_ref[1], wr[LANES:])
         + bl_ref[...])
    h = jnp.maximum(h, 0.0)
    o_ref[...] = _dot(h, ow_ref[...]) + ob_ref[...]


def _halves_spec():
    return pl.BlockSpec((NCORES, BN, LANES), lambda i: (0, i, 0))


def _full_spec(shape):
    return pl.BlockSpec(shape, lambda i: tuple(0 for _ in shape))


def kernel(x_content, x_style, edge_index, edge_type, post_W, post_b,
           style_W, style_b, Wl1, bl1, Wr1, Wl2, bl2, Wr2, Wl3, bl3, Wr3,
           out_W, out_b):
    N = x_content.shape[0]
    E = edge_index.shape[1]
    H = Wl1.shape[1]
    NC = out_W.shape[1]
    grid = (N // BN,)

    src = edge_index[0]
    dst = edge_index[1]

    deg_kernel = _make_deg_kernel(N, E)
    agg_kernel = _make_agg_kernel(N, E)

    D = deg_kernel(dst)

    X1 = pl.pallas_call(
        _encode_body,
        grid=grid,
        in_specs=[
            pl.BlockSpec((BN, x_content.shape[1]), lambda i: (i, 0)),
            pl.BlockSpec((BN, x_style.shape[1]), lambda i: (i, 0)),
            _full_spec(post_W.shape),
            _full_spec((1, LANES)),
            _full_spec(style_W.shape),
            _full_spec((1, LANES)),
        ],
        out_specs=_halves_spec(),
        out_shape=jax.ShapeDtypeStruct((NCORES, N, LANES), jnp.float32),
    )(x_content, x_style, post_W, post_b.reshape(1, -1),
      style_W, style_b.reshape(1, -1))

    def combine(X, A, Wl, bl, Wr):
        return pl.pallas_call(
            _combine_body,
            grid=grid,
            in_specs=[
                _halves_spec(),
                _halves_spec(),
                _halves_spec(),
                _full_spec(Wl.shape),
                _full_spec((1, H)),
                _full_spec(Wr.shape),
            ],
            out_specs=_halves_spec(),
            out_shape=jax.ShapeDtypeStruct((NCORES, N, LANES), jnp.float32),
        )(X, A, D, Wl, bl.reshape(1, -1), Wr)

    A1 = agg_kernel(X1, src, dst)
    X2 = combine(X1, A1, Wl1, bl1, Wr1)
    A2 = agg_kernel(X2, src, dst)
    X3 = combine(X2, A2, Wl2, bl2, Wr2)
    A3 = agg_kernel(X3, src, dst)

    out = pl.pallas_call(
        _final_body,
        grid=grid,
        in_specs=[
            _halves_spec(),
            _halves_spec(),
            _halves_spec(),
            _full_spec(Wl3.shape),
            _full_spec((1, H)),
            _full_spec(Wr3.shape),
            _full_spec(out_W.shape),
            _full_spec((1, NC)),
        ],
        out_specs=pl.BlockSpec((BN, NC), lambda i: (i, 0)),
        out_shape=jax.ShapeDtypeStruct((N, NC), jnp.float32),
    )(X3, A3, D, Wl3, bl3.reshape(1, -1), Wr3, out_W, out_b.reshape(1, -1))

    return out


# speedup vs baseline: 4.6092x; 4.6092x over previous
import functools

import jax
import jax.numpy as jnp
from jax import lax
from jax.experimental import pallas as pl
from jax.experimental.pallas import tpu as pltpu
from jax.experimental.pallas import tpu_sc as plsc

NCORES = 2
NSUB = 16
LANES = 16
W = 128
BN = 2000


def _mesh():
    return plsc.VectorSubcoreMesh(core_axis_name="c", subcore_axis_name="s")


def _make_deg_kernel(NP, E):
    EPC = E // (NCORES * NSUB)
    NW = EPC // W
    TAIL = EPC - NW * W
    SR = NP // NSUB
    ZR = max(d for d in range(8, 1025, 8) if SR % d == 0)

    @functools.partial(
        pl.kernel,
        mesh=_mesh(),
        compiler_params=pltpu.CompilerParams(use_tc_tiling_on_sc=False),
        out_type=jax.ShapeDtypeStruct((NCORES, NP, LANES), jnp.float32),
        scratch_types=[
            pltpu.VMEM_SHARED((NP, LANES), jnp.float32),
            pltpu.VMEM((ZR, LANES), jnp.float32),
            pltpu.VMEM((W, LANES), jnp.float32),
            pltpu.VMEM((W,), jnp.int32),
        ]
        + ([pltpu.VMEM((TAIL, LANES), jnp.float32),
            pltpu.VMEM((TAIL,), jnp.int32)] if TAIL else []),
    )
    def deg_kernel(dst_hbm, out_hbm, acc, zb, ones, dst_v, *tail_bufs):
        c = lax.axis_index("c")
        s = lax.axis_index("s")

        @pl.loop(0, ZR)
        def _(i):
            zb[i, :] = jnp.zeros((LANES,), jnp.float32)

        @pl.loop(0, W)
        def _(i):
            ones[i, :] = jnp.ones((LANES,), jnp.float32)

        @pl.loop(0, SR // ZR)
        def _(j):
            pltpu.sync_copy(zb, acc.at[pl.ds(s * SR + j * ZR, ZR)])

        plsc.subcore_barrier()

        base = c * (E // NCORES) + s * EPC

        @pl.loop(0, NW)
        def _(w):
            pltpu.sync_copy(dst_hbm.at[pl.ds(base + w * W, W)], dst_v)
            pltpu.sync_copy(ones, acc.at[dst_v], add=True)

        if TAIL:
            ones_t, dst_t = tail_bufs

            @pl.loop(0, TAIL)
            def _(i):
                ones_t[i, :] = jnp.ones((LANES,), jnp.float32)

            pltpu.sync_copy(dst_hbm.at[pl.ds(base + NW * W, TAIL)], dst_t)
            pltpu.sync_copy(ones_t, acc.at[dst_t], add=True)

        plsc.subcore_barrier()
        pltpu.sync_copy(acc.at[pl.ds(s * SR, SR)],
                        out_hbm.at[c].at[pl.ds(s * SR, SR)])

    return deg_kernel


def _make_agg_kernel(NP, E):
    EPC = E // NSUB
    NW = EPC // W
    TAIL = EPC - NW * W
    SR = NP // NSUB
    ZR = max(d for d in range(8, 1025, 8) if SR % d == 0)

    @functools.partial(
        pl.kernel,
        mesh=_mesh(),
        compiler_params=pltpu.CompilerParams(use_tc_tiling_on_sc=False),
        out_type=jax.ShapeDtypeStruct((NCORES, NP, LANES), jnp.float32),
        scratch_types=[
            pltpu.VMEM_SHARED((NP, LANES), jnp.float32),
            pltpu.VMEM((ZR, LANES), jnp.float32),
            pltpu.VMEM((W, LANES), jnp.float32),
            pltpu.VMEM((W,), jnp.int32),
            pltpu.VMEM((W,), jnp.int32),
        ]
        + ([pltpu.VMEM((TAIL, LANES), jnp.float32),
            pltpu.VMEM((TAIL,), jnp.int32),
            pltpu.VMEM((TAIL,), jnp.int32)] if TAIL else []),
    )
    def agg_kernel(x_hbm, src_hbm, dst_hbm, out_hbm,
                   acc, zb, msg, src_v, dst_v, *tail_bufs):
        c = lax.axis_index("c")
        s = lax.axis_index("s")

        @pl.loop(0, ZR)
        def _(i):
            zb[i, :] = jnp.zeros((LANES,), jnp.float32)

        @pl.loop(0, SR // ZR)
        def _(j):
            pltpu.sync_copy(zb, acc.at[pl.ds(s * SR + j * ZR, ZR)])

        plsc.subcore_barrier()

        base = s * EPC

        @pl.loop(0, NW)
        def _(w):
            off = base + w * W
            pltpu.sync_copy(src_hbm.at[pl.ds(off, W)], src_v)
            pltpu.sync_copy(dst_hbm.at[pl.ds(off, W)], dst_v)
            pltpu.sync_copy(x_hbm.at[c].at[src_v], msg)
            pltpu.sync_copy(msg, acc.at[dst_v], add=True)

        if TAIL:
            msg_t, src_t, dst_t = tail_bufs
            off = base + NW * W
            pltpu.sync_copy(src_hbm.at[pl.ds(off, TAIL)], src_t)
            pltpu.sync_copy(dst_hbm.at[pl.ds(off, TAIL)], dst_t)
            pltpu.sync_copy(x_hbm.at[c].at[src_t], msg_t)
            pltpu.sync_copy(msg_t, acc.at[dst_t], add=True)

        plsc.subcore_barrier()
        pltpu.sync_copy(acc.at[pl.ds(s * SR, SR)],
                        out_hbm.at[c].at[pl.ds(s * SR, SR)])

    return agg_kernel


def _dot(a, b):
    return jnp.dot(a, b, preferred_element_type=jnp.float32)


def _encode_body(xc_ref, xs_ref, pw_ref, pb_ref, sw_ref, sb_ref, o_ref):
    o_ref[0] = _dot(xc_ref[...], pw_ref[...]) + pb_ref[...]
    o_ref[1] = _dot(xs_ref[...], sw_ref[...]) + sb_ref[...]


def _combine_body(x_ref, a_ref, d_ref, wl_ref, bl_ref, wr_ref, o_ref):
    wl = wl_ref[...]
    wr = wr_ref[...]
    d = d_ref[0, :, 0:1] + d_ref[1, :, 0:1]
    invd = 1.0 / jnp.maximum(d, 1.0)
    h = (_dot(a_ref[0] * invd, wl[:LANES])
         + _dot(a_ref[1] * invd, wl[LANES:])
         + _dot(x_ref[0], wr[:LANES])
         + _dot(x_ref[1], wr[LANES:])
         + bl_ref[...])
    h = jnp.maximum(h, 0.0)
    o_ref[0] = h[:, :LANES]
    o_ref[1] = h[:, LANES:]


def _final_body(x_ref, a_ref, d_ref, wl_ref, bl_ref, wr_ref, ow_ref, ob_ref,
                o_ref):
    wl = wl_ref[...]
    wr = wr_ref[...]
    d = d_ref[0, :, 0:1] + d_ref[1, :, 0:1]
    invd = 1.0 / jnp.maximum(d, 1.0)
    h = (_dot(a_ref[0] * invd, wl[:LANES])
         + _dot(a_ref[1] * invd, wl[LANES:])
         + _dot(x_ref[0], wr[:LANES])
         + _dot(x_ref[1], wr[LANES:])
         + bl_ref[...])
    h = jnp.maximum(h, 0.0)
    o_ref[...] = _dot(h, ow_ref[...]) + ob_ref[...]


def _halves_spec():
    return pl.BlockSpec((NCORES, BN, LANES), lambda i: (0, i, 0))


def _full_spec(shape):
    return pl.BlockSpec(shape, lambda i: tuple(0 for _ in shape))


def kernel(x_content, x_style, edge_index, edge_type, post_W, post_b,
           style_W, style_b, Wl1, bl1, Wr1, Wl2, bl2, Wr2, Wl3, bl3, Wr3,
           out_W, out_b):
    N = x_content.shape[0]
    E = edge_index.shape[1]
    H = Wl1.shape[1]
    NC = out_W.shape[1]
    NP = -(-N // (8 * NSUB)) * (8 * NSUB)
    grid = (N // BN,)

    src = edge_index[0]
    dst = edge_index[1]

    deg_kernel = _make_deg_kernel(NP, E)
    agg_kernel = _make_agg_kernel(NP, E)

    D = deg_kernel(dst)

    X1 = pl.pallas_call(
        _encode_body,
        grid=grid,
        in_specs=[
            pl.BlockSpec((BN, x_content.shape[1]), lambda i: (i, 0)),
            pl.BlockSpec((BN, x_style.shape[1]), lambda i: (i, 0)),
            _full_spec(post_W.shape),
            _full_spec((1, LANES)),
            _full_spec(style_W.shape),
            _full_spec((1, LANES)),
        ],
        out_specs=_halves_spec(),
        out_shape=jax.ShapeDtypeStruct((NCORES, NP, LANES), jnp.float32),
    )(x_content, x_style, post_W, post_b.reshape(1, -1),
      style_W, style_b.reshape(1, -1))

    def combine(X, A, Wl, bl, Wr):
        return pl.pallas_call(
            _combine_body,
            grid=grid,
            in_specs=[
                _halves_spec(),
                _halves_spec(),
                _halves_spec(),
                _full_spec(Wl.shape),
                _full_spec((1, H)),
                _full_spec(Wr.shape),
            ],
            out_specs=_halves_spec(),
            out_shape=jax.ShapeDtypeStruct((NCORES, NP, LANES), jnp.float32),
        )(X, A, D, Wl, bl.reshape(1, -1), Wr)

    A1 = agg_kernel(X1, src, dst)
    X2 = combine(X1, A1, Wl1, bl1, Wr1)
    A2 = agg_kernel(X2, src, dst)
    X3 = combine(X2, A2, Wl2, bl2, Wr2)
    A3 = agg_kernel(X3, src, dst)

    out = pl.pallas_call(
        _final_body,
        grid=grid,
        in_specs=[
            _halves_spec(),
            _halves_spec(),
            _halves_spec(),
            _full_spec(Wl3.shape),
            _full_spec((1, H)),
            _full_spec(Wr3.shape),
            _full_spec(out_W.shape),
            _full_spec((1, NC)),
        ],
        out_specs=pl.BlockSpec((BN, NC), lambda i: (i, 0)),
        out_shape=jax.ShapeDtypeStruct((N, NC), jnp.float32),
    )(X3, A3, D, Wl3, bl3.reshape(1, -1), Wr3, out_W, out_b.reshape(1, -1))

    return out


# speedup vs baseline: 16.0756x; 3.4877x over previous
import functools

import jax
import jax.numpy as jnp
from jax import lax
from jax.experimental import pallas as pl
from jax.experimental.pallas import tpu as pltpu
from jax.experimental.pallas import tpu_sc as plsc

NCORES = 2
NSUB = 16
LANES = 16
W = 128
BN = 2000


def _mesh():
    return plsc.VectorSubcoreMesh(core_axis_name="c", subcore_axis_name="s")


def _make_deg_kernel(NP, NRP):
    NW = NRP // (NCORES * NSUB)
    SR = NP // NSUB
    ZR = max(d for d in range(8, 513, 8) if SR % d == 0)
    D = 8
    L = 4

    @functools.partial(
        pl.kernel,
        mesh=_mesh(),
        compiler_params=pltpu.CompilerParams(use_tc_tiling_on_sc=False),
        out_type=jax.ShapeDtypeStruct((NCORES, NP, LANES), jnp.float32),
        scratch_types=[
            pltpu.VMEM_SHARED((NP, LANES), jnp.float32),
            pltpu.VMEM((ZR, LANES), jnp.float32),
            pltpu.VMEM((W, LANES), jnp.float32),
            pltpu.VMEM((D, W), jnp.int32),
            pltpu.SemaphoreType.DMA((D,)),
            pltpu.SemaphoreType.DMA((D,)),
        ],
    )
    def deg_kernel(dst_hbm, out_hbm, acc, zb, ones, dbuf, semI, semS):
        c = lax.axis_index("c")
        s = lax.axis_index("s")

        @pl.loop(0, ZR)
        def _(i):
            zb[i, :] = jnp.zeros((LANES,), jnp.float32)

        @pl.loop(0, W)
        def _(i):
            ones[i, :] = jnp.ones((LANES,), jnp.float32)

        @pl.loop(0, SR // ZR)
        def _(j):
            pltpu.sync_copy(zb, acc.at[pl.ds(s * SR + j * ZR, ZR)])

        plsc.subcore_barrier()

        base = (c * NSUB + s) * NW

        @pl.loop(0, NW + D)
        def _(k):
            m = lax.rem(k, D)

            @pl.when(k >= D)
            def _():
                pltpu.make_async_copy(dst_hbm.at[base], ones,
                                      semS.at[m]).wait()

            @pl.when(k < NW)
            def _():
                pltpu.async_copy(dst_hbm.at[base + k], dbuf.at[m],
                                 semI.at[m])

            @pl.when((k >= L) & (k - L < NW))
            def _():
                ml = lax.rem(k - L, D)
                pltpu.make_async_copy(dst_hbm.at[base], dbuf.at[ml],
                                      semI.at[ml]).wait()
                pltpu.async_copy(ones, acc.at[dbuf.at[ml]], semS.at[ml],
                                 add=True)

        plsc.subcore_barrier()
        pltpu.sync_copy(acc.at[pl.ds(s * SR, SR)],
                        out_hbm.at[c].at[pl.ds(s * SR, SR)])

    return deg_kernel


def _make_agg_kernel(NP, NRP):
    NW = NRP // NSUB
    SR = NP // NSUB
    ZR = max(d for d in range(8, 513, 8) if SR % d == 0)
    D = 10
    L1 = 3
    L2 = 3

    @functools.partial(
        pl.kernel,
        mesh=_mesh(),
        compiler_params=pltpu.CompilerParams(use_tc_tiling_on_sc=False),
        out_type=jax.ShapeDtypeStruct((NCORES, NP, LANES), jnp.float32),
        scratch_types=[
            pltpu.VMEM_SHARED((NP, LANES), jnp.float32),
            pltpu.VMEM((ZR, LANES), jnp.float32),
            pltpu.VMEM((D, W, LANES), jnp.float32),
            pltpu.VMEM((D, W), jnp.int32),
            pltpu.VMEM((D, W), jnp.int32),
            pltpu.SemaphoreType.DMA((D,)),
            pltpu.SemaphoreType.DMA((D,)),
            pltpu.SemaphoreType.DMA((D,)),
        ],
    )
    def agg_kernel(x_hbm, src_hbm, dst_hbm, out_hbm,
                   acc, zb, msg, sbuf, dbuf, semI, semG, semS):
        c = lax.axis_index("c")
        s = lax.axis_index("s")

        @pl.loop(0, ZR)
        def _(i):
            zb[i, :] = jnp.zeros((LANES,), jnp.float32)

        @pl.loop(0, SR // ZR)
        def _(j):
            pltpu.sync_copy(zb, acc.at[pl.ds(s * SR + j * ZR, ZR)])

        plsc.subcore_barrier()

        base = s * NW

        @pl.loop(0, NW + D)
        def _(k):
            m = lax.rem(k, D)

            @pl.when(k >= D)
            def _():
                pltpu.make_async_copy(x_hbm.at[c].at[pl.ds(0, W)],
                                      msg.at[m], semS.at[m]).wait()

            @pl.when(k < NW)
            def _():
                pltpu.async_copy(src_hbm.at[base + k], sbuf.at[m],
                                 semI.at[m])
                pltpu.async_copy(dst_hbm.at[base + k], dbuf.at[m],
                                 semI.at[m])

            @pl.when((k >= L1) & (k - L1 < NW))
            def _():
                m1 = lax.rem(k - L1, D)
                pltpu.make_async_copy(src_hbm.at[base], sbuf.at[m1],
                                      semI.at[m1]).wait()
                pltpu.make_async_copy(dst_hbm.at[base], dbuf.at[m1],
                                      semI.at[m1]).wait()
                pltpu.async_copy(x_hbm.at[c].at[sbuf.at[m1]], msg.at[m1],
                                 semG.at[m1])

            @pl.when((k >= L1 + L2) & (k - L1 - L2 < NW))
            def _():
                m2 = lax.rem(k - L1 - L2, D)
                pltpu.make_async_copy(x_hbm.at[c].at[pl.ds(0, W)],
                                      msg.at[m2], semG.at[m2]).wait()
                pltpu.async_copy(msg.at[m2], acc.at[dbuf.at[m2]],
                                 semS.at[m2], add=True)

        plsc.subcore_barrier()
        pltpu.sync_copy(acc.at[pl.ds(s * SR, SR)],
                        out_hbm.at[c].at[pl.ds(s * SR, SR)])

    return agg_kernel


def _dot(a, b):
    return jnp.dot(a, b, preferred_element_type=jnp.float32)


def _encode_body(xc_ref, xs_ref, pw_ref, pb_ref, sw_ref, sb_ref, o_ref):
    o_ref[0] = _dot(xc_ref[...], pw_ref[...]) + pb_ref[...]
    o_ref[1] = _dot(xs_ref[...], sw_ref[...]) + sb_ref[...]


def _combine_body(x_ref, a_ref, d_ref, wl_ref, bl_ref, wr_ref, o_ref):
    wl = wl_ref[...]
    wr = wr_ref[...]
    d = d_ref[0, :, 0:1] + d_ref[1, :, 0:1]
    invd = 1.0 / jnp.maximum(d, 1.0)
    h = (_dot(a_ref[0] * invd, wl[:LANES])
         + _dot(a_ref[1] * invd, wl[LANES:])
         + _dot(x_ref[0], wr[:LANES])
         + _dot(x_ref[1], wr[LANES:])
         + bl_ref[...])
    h = jnp.maximum(h, 0.0)
    o_ref[0] = h[:, :LANES]
    o_ref[1] = h[:, LANES:]


def _final_body(x_ref, a_ref, d_ref, wl_ref, bl_ref, wr_ref, ow_ref, ob_ref,
                o_ref):
    wl = wl_ref[...]
    wr = wr_ref[...]
    d = d_ref[0, :, 0:1] + d_ref[1, :, 0:1]
    invd = 1.0 / jnp.maximum(d, 1.0)
    h = (_dot(a_ref[0] * invd, wl[:LANES])
         + _dot(a_ref[1] * invd, wl[LANES:])
         + _dot(x_ref[0], wr[:LANES])
         + _dot(x_ref[1], wr[LANES:])
         + bl_ref[...])
    h = jnp.maximum(h, 0.0)
    o_ref[...] = _dot(h, ow_ref[...]) + ob_ref[...]


def _halves_spec():
    return pl.BlockSpec((NCORES, BN, LANES), lambda i: (0, i, 0))


def _full_spec(shape):
    return pl.BlockSpec(shape, lambda i: tuple(0 for _ in shape))


def kernel(x_content, x_style, edge_index, edge_type, post_W, post_b,
           style_W, style_b, Wl1, bl1, Wr1, Wl2, bl2, Wr2, Wl3, bl3, Wr3,
           out_W, out_b):
    N = x_content.shape[0]
    E = edge_index.shape[1]
    H = Wl1.shape[1]
    NC = out_W.shape[1]
    NP = -(-N // (8 * NSUB)) * (8 * NSUB)
    grid = (N // BN,)

    NRP = -(-E // (W * 2 * NSUB)) * (2 * NSUB)
    EP = NRP * W
    src = jnp.concatenate(
        [edge_index[0], jnp.zeros((EP - E,), jnp.int32)]).reshape(NRP, W)
    dst = jnp.concatenate(
        [edge_index[1], jnp.full((EP - E,), N, jnp.int32)]).reshape(NRP, W)

    deg_kernel = _make_deg_kernel(NP, NRP)
    agg_kernel = _make_agg_kernel(NP, NRP)

    D = deg_kernel(dst)

    X1 = pl.pallas_call(
        _encode_body,
        grid=grid,
        in_specs=[
            pl.BlockSpec((BN, x_content.shape[1]), lambda i: (i, 0)),
            pl.BlockSpec((BN, x_style.shape[1]), lambda i: (i, 0)),
            _full_spec(post_W.shape),
            _full_spec((1, LANES)),
            _full_spec(style_W.shape),
            _full_spec((1, LANES)),
        ],
        out_specs=_halves_spec(),
        out_shape=jax.ShapeDtypeStruct((NCORES, NP, LANES), jnp.float32),
    )(x_content, x_style, post_W, post_b.reshape(1, -1),
      style_W, style_b.reshape(1, -1))

    def combine(X, A, Wl, bl, Wr):
        return pl.pallas_call(
            _combine_body,
            grid=grid,
            in_specs=[
                _halves_spec(),
                _halves_spec(),
                _halves_spec(),
                _full_spec(Wl.shape),
                _full_spec((1, H)),
                _full_spec(Wr.shape),
            ],
            out_specs=_halves_spec(),
            out_shape=jax.ShapeDtypeStruct((NCORES, NP, LANES), jnp.float32),
        )(X, A, D, Wl, bl.reshape(1, -1), Wr)

    A1 = agg_kernel(X1, src, dst)
    X2 = combine(X1, A1, Wl1, bl1, Wr1)
    A2 = agg_kernel(X2, src, dst)
    X3 = combine(X2, A2, Wl2, bl2, Wr2)
    A3 = agg_kernel(X3, src, dst)

    out = pl.pallas_call(
        _final_body,
        grid=grid,
        in_specs=[
            _halves_spec(),
            _halves_spec(),
            _halves_spec(),
            _full_spec(Wl3.shape),
            _full_spec((1, H)),
            _full_spec(Wr3.shape),
            _full_spec(out_W.shape),
            _full_spec((1, NC)),
        ],
        out_specs=pl.BlockSpec((BN, NC), lambda i: (i, 0)),
        out_shape=jax.ShapeDtypeStruct((N, NC), jnp.float32),
    )(X3, A3, D, Wl3, bl3.reshape(1, -1), Wr3, out_W, out_b.reshape(1, -1))

    return out
